# baseline (speedup 1.0000x reference)
import functools

import jax
import jax.numpy as jnp
from jax import lax
from jax.experimental import pallas as pl
from jax.experimental.pallas import tpu as pltpu
from jax.experimental.pallas import tpu_sc as plsc

_CB = 16


def _values_body(x_ref, val_ref):
    cb = pl.program_id(1)
    h = val_ref.shape[2]
    w = val_ref.shape[3]
    val_ref[...] = jnp.zeros(val_ref.shape, jnp.float32)

    @pl.when(cb == 0)
    def _():
        row = jax.lax.broadcasted_iota(jnp.int32, (h, w), 0)
        col = jax.lax.broadcasted_iota(jnp.int32, (h, w), 1)
        val_ref[0, 0] = jnp.where(row == col, x_ref[0, 0], 0.0)


def kernel(data_batch):
    B, C, H, W = data_batch.shape
    D = min(H, W)
    NC, NS = 2, 16
    NW = NC * NS
    RPW = (B * D) // NW

    values = pl.pallas_call(
        _values_body,
        grid=(B, C // _CB),
        in_specs=[pl.BlockSpec((1, 1, H, W), lambda b, c: (b, 0, 0, 0))],
        out_specs=pl.BlockSpec((1, _CB, H, W), lambda b, c: (b, c, 0, 0)),
        out_shape=jax.ShapeDtypeStruct((B, C, H, W), jnp.float32),
        compiler_params=pltpu.CompilerParams(
            dimension_semantics=("arbitrary", "arbitrary"),
        ),
    )(data_batch)

    mesh = plsc.VectorSubcoreMesh(core_axis_name="c", subcore_axis_name="s")

    @functools.partial(
        pl.kernel,
        mesh=mesh,
        out_type=jax.ShapeDtypeStruct((B * D * 4,), jnp.int32),
        scratch_types=[pltpu.VMEM((RPW * 4,), jnp.int32)],
    )
    def sc_indices(ind_hbm, indb):
        wid = lax.axis_index("s") * NC + lax.axis_index("c")
        lane = lax.broadcasted_iota(jnp.int32, (16,), 0)
        bvec = lax.broadcast_in_dim(wid >> 2, (16,), ())
        ivec = lax.broadcast_in_dim((wid & 3) * RPW, (16,), ())
        zero16 = jnp.zeros((16,), jnp.int32)
        for t in range(RPW * 4 // 16):
            e = t * 16 + lane
            k = e >> 2
            col = e & 3
            v = jnp.where(col == 0, bvec, jnp.where(col == 1, zero16, ivec + k))
            indb[pl.ds(t * 16, 16)] = v
        pltpu.sync_copy(indb, ind_hbm.at[pl.ds(wid * RPW * 4, RPW * 4)])

    indices = sc_indices().reshape(B * D, 4)

    return (values, indices)


# speedup vs baseline: 1.0121x; 1.0121x over previous
import functools

import jax
import jax.numpy as jnp
from jax import lax
from jax.experimental import pallas as pl
from jax.experimental.pallas import tpu as pltpu
from jax.experimental.pallas import tpu_sc as plsc

_CB = 16


def _values_body(x_ref, val_ref):
    cb = pl.program_id(1)
    h = val_ref.shape[2]
    w = val_ref.shape[3]
    val_ref[...] = jnp.zeros(val_ref.shape, jnp.float32)

    @pl.when(cb == 0)
    def _():
        row = jax.lax.broadcasted_iota(jnp.int32, (h, w), 0)
        col = jax.lax.broadcasted_iota(jnp.int32, (h, w), 1)
        val_ref[0, 0] = jnp.where(row == col, x_ref[0, 0], 0.0)


def kernel(data_batch):
    B, C, H, W = data_batch.shape
    D = min(H, W)
    NC, NS = 1, 16
    NW = NC * NS
    RPW = (B * D) // NW

    values = pl.pallas_call(
        _values_body,
        grid=(B, C // _CB),
        in_specs=[pl.BlockSpec((1, 1, H, W), lambda b, c: (b, 0, 0, 0))],
        out_specs=pl.BlockSpec((1, _CB, H, W), lambda b, c: (b, c, 0, 0)),
        out_shape=jax.ShapeDtypeStruct((B, C, H, W), jnp.float32),
        compiler_params=pltpu.CompilerParams(
            dimension_semantics=("arbitrary", "arbitrary"),
        ),
    )(data_batch)

    mesh = plsc.VectorSubcoreMesh(core_axis_name="c", subcore_axis_name="s", num_cores=1)

    @functools.partial(
        pl.kernel,
        mesh=mesh,
        out_type=jax.ShapeDtypeStruct((B * D * 4,), jnp.int32),
        scratch_types=[pltpu.VMEM((RPW * 4,), jnp.int32)],
    )
    def sc_indices(ind_hbm, indb):
        wid = lax.axis_index("s") * NC + lax.axis_index("c")
        lane = lax.broadcasted_iota(jnp.int32, (16,), 0)
        wpb_shift = (NW // B).bit_length() - 1
        bvec = lax.broadcast_in_dim(wid >> wpb_shift, (16,), ())
        ivec = lax.broadcast_in_dim((wid & (NW // B - 1)) * RPW, (16,), ())
        zero16 = jnp.zeros((16,), jnp.int32)
        for t in range(RPW * 4 // 16):
            e = t * 16 + lane
            k = e >> 2
            col = e & 3
            v = jnp.where(col == 0, bvec, jnp.where(col == 1, zero16, ivec + k))
            indb[pl.ds(t * 16, 16)] = v
        pltpu.sync_copy(indb, ind_hbm.at[pl.ds(wid * RPW * 4, RPW * 4)])

    indices = sc_indices().reshape(B * D, 4)

    return (values, indices)


# speedup vs baseline: 1.1421x; 1.1284x over previous
import jax
import jax.numpy as jnp
from jax.experimental import pallas as pl
from jax.experimental.pallas import tpu as pltpu

_CB = 16


def _body(x_ref, val_ref, ind_ref):
    b = pl.program_id(0)
    cb = pl.program_id(1)
    h = val_ref.shape[2]
    w = val_ref.shape[3]
    val_ref[...] = jnp.zeros(val_ref.shape, jnp.float32)

    @pl.when(cb == 0)
    def _():
        row = jax.lax.broadcasted_iota(jnp.int32, (h, w), 0)
        col = jax.lax.broadcasted_iota(jnp.int32, (h, w), 1)
        val_ref[0, 0] = jnp.where(row == col, x_ref[0, 0], 0.0)

    @pl.when((b == 0) & (cb == 0))
    def _():
        n = ind_ref.shape[1]
        d = min(h, w)
        r = jax.lax.broadcasted_iota(jnp.int32, (4, n), 1)
        c = jax.lax.broadcasted_iota(jnp.int32, (4, n), 0)
        ind_ref[...] = jnp.where(c == 0, r // d, jnp.where(c == 1, 0, r % d))


def kernel(data_batch):
    B, C, H, W = data_batch.shape
    D = min(H, W)

    values, indices_t = pl.pallas_call(
        _body,
        grid=(B, C // _CB),
        in_specs=[pl.BlockSpec((1, 1, H, W), lambda b, c: (b, 0, 0, 0))],
        out_specs=[
            pl.BlockSpec((1, _CB, H, W), lambda b, c: (b, c, 0, 0)),
            pl.BlockSpec((4, B * D), lambda b, c: (0, 0)),
        ],
        out_shape=[
            jax.ShapeDtypeStruct((B, C, H, W), jnp.float32),
            jax.ShapeDtypeStruct((4, B * D), jnp.int32),
        ],
        compiler_params=pltpu.CompilerParams(
            dimension_semantics=("arbitrary", "arbitrary"),
        ),
    )(data_batch)

    return (values, indices_t.T)
